# baseline (speedup 1.0000x reference)
import jax
import jax.numpy as jnp
from jax import lax
from jax.experimental import pallas as pl
from jax.experimental.pallas import tpu as pltpu
from jax.experimental.pallas import tpu_sc as plsc

N_ROWS = 16384
N_COLS = 50
EMB_DIM = 32
LANES = 16

NC = 2
NS = 16
NW = NC * NS
N_PAIRS = N_COLS * (N_ROWS // 128)
PAIRS_W = N_PAIRS // NW
GB = 8
TB = 4


def _transpose16(regs, perm, masks):
    for k in range(4):
        d = 1 << k
        out = list(regs)
        for i in range(LANES):
            if i & d:
                continue
            j = i | d
            a, b = regs[i], regs[j]
            bp = b.at[perm[k]].get(mode="promise_in_bounds")
            ap = a.at[perm[k]].get(mode="promise_in_bounds")
            out[i] = jnp.where(masks[k], bp, a)
            out[j] = jnp.where(masks[k], b, ap)
        regs = out
    return regs


def _body(idx_hbm, w_hbm, out_hbm, idx_v, bufs, trans, sem_g, sem_o):
    wid = lax.axis_index("s") * NC + lax.axis_index("c")
    p0 = wid * PAIRS_W
    pltpu.sync_copy(idx_hbm.at[pl.ds(p0, PAIRS_W)], idx_v)

    lane = lax.iota(jnp.int32, LANES)
    perm = [lax.bitwise_xor(lane, 1 << k) for k in range(4)]
    masks = [lax.bitwise_and(lane, 1 << k) != 0 for k in range(4)]

    for i in range(GB):
        pltpu.async_copy(
            w_hbm.at[idx_v.at[i]], bufs.at[pl.ds(i * 128, 128)], sem_g.at[i]
        )

    def pair_step(t, carry):
        b = lax.bitwise_and(t, GB - 1)
        tb = lax.bitwise_and(t, TB - 1)
        p = p0 + t
        c = lax.shift_right_logical(p, 7)
        j = lax.bitwise_and(p, 127)
        brow = b * 128

        pltpu.make_async_copy(
            w_hbm.at[pl.ds(0, 128)], bufs.at[pl.ds(brow, 128)], sem_g.at[b]
        ).wait()

        @pl.when(t >= TB)
        def _():
            for a in range(EMB_DIM // 8):
                pltpu.make_async_copy(
                    out_hbm.at[0, 0, 0],
                    trans.at[tb, a],
                    sem_o.at[tb],
                ).wait()

        for bt in range(128 // LANES):
            for h in range(EMB_DIM // LANES):
                regs = [
                    bufs[brow + bt * LANES + i, pl.ds(h * LANES, LANES)]
                    for i in range(LANES)
                ]
                regs = _transpose16(regs, perm, masks)
                for fl in range(LANES):
                    f = h * LANES + fl
                    trans[tb, f >> 3, f & 7, pl.ds(bt * LANES, LANES)] = regs[fl]

        for a in range(EMB_DIM // 8):
            pltpu.async_copy(
                trans.at[tb, a],
                out_hbm.at[c, a, j],
                sem_o.at[tb],
            )

        @pl.when(t + GB < PAIRS_W)
        def _():
            pltpu.async_copy(
                w_hbm.at[idx_v.at[t + GB]],
                bufs.at[pl.ds(brow, 128)],
                sem_g.at[b],
            )

        return carry

    lax.fori_loop(0, PAIRS_W, pair_step, 0)

    for tslot in range(TB):
        for a in range(EMB_DIM // 8):
            pltpu.make_async_copy(
                out_hbm.at[0, 0, 0],
                trans.at[tslot, a],
                sem_o.at[tslot],
            ).wait()


NUM_EMB = 1000000
TC_BLK = 8192
QUART = TC_BLK // 4
N_BLK = (NUM_EMB + TC_BLK - 1) // TC_BLK
NUM_EMB_PAD = N_BLK * TC_BLK


def _tc_transpose_body(wt_ref, out_ref):
    eye = jnp.eye(EMB_DIM, dtype=jnp.float32)
    for q in range(4):
        out_ref[:, q * EMB_DIM:(q + 1) * EMB_DIM] = lax.dot_general(
            wt_ref[:, q * QUART:(q + 1) * QUART],
            eye,
            (((0,), (0,)), ((), ())),
            preferred_element_type=jnp.float32,
        )


def _to_row_major(weight):
    wt = weight.T
    return pl.pallas_call(
        _tc_transpose_body,
        grid=(N_BLK,),
        in_specs=[pl.BlockSpec((EMB_DIM, TC_BLK), lambda j: (0, j))],
        out_specs=pl.BlockSpec((QUART, 128), lambda j: (j, 0)),
        out_shape=jax.ShapeDtypeStruct(
            (NUM_EMB_PAD * EMB_DIM // 128, 128), jnp.float32
        ),
    )(wt)


def _swizzle(tid):
    blk = jnp.bitwise_and(tid, jnp.int32(~(TC_BLK - 1)))
    r = jnp.bitwise_and(tid, QUART - 1)
    q = jnp.bitwise_and(jnp.right_shift(tid, 11), 3)
    return blk + jnp.left_shift(r, 2) + q


@jax.jit
def _gather(idx, weight):
    mesh = plsc.VectorSubcoreMesh(core_axis_name="c", subcore_axis_name="s")
    fn = pl.kernel(
        _body,
        out_type=jax.ShapeDtypeStruct(
            (N_COLS, EMB_DIM // 8, N_ROWS // 128, 8, 128), jnp.float32
        ),
        mesh=mesh,
        scratch_types=[
            pltpu.VMEM((PAIRS_W, 128), jnp.int32),
            pltpu.VMEM((GB * 128, EMB_DIM), jnp.float32),
            pltpu.VMEM((TB, EMB_DIM // 8, 8, 128), jnp.float32),
            pltpu.SemaphoreType.DMA((GB,)),
            pltpu.SemaphoreType.DMA((TB,)),
        ],
        compiler_params=pltpu.CompilerParams(use_tc_tiling_on_sc=False),
    )
    return fn(idx, weight)


def kernel(token_ids, weight):
    tid2 = _swizzle(token_ids.astype(jnp.int32)).T.reshape(N_PAIRS, 128)
    w_rm = _to_row_major(weight).reshape(NUM_EMB_PAD, EMB_DIM)
    out5 = _gather(tid2, w_rm)
    return out5.transpose(2, 4, 0, 1, 3).reshape(N_ROWS, N_COLS, EMB_DIM)


# speedup vs baseline: 1.0180x; 1.0180x over previous
import jax
import jax.numpy as jnp
from jax import lax
from jax.experimental import pallas as pl
from jax.experimental.pallas import tpu as pltpu
from jax.experimental.pallas import tpu_sc as plsc

N_ROWS = 16384
N_COLS = 50
EMB_DIM = 32
LANES = 16

NC = 2
NS = 16
NW = NC * NS
N_PAIRS = N_COLS * (N_ROWS // 128)
PAIRS_W = N_PAIRS // NW
GB = 8
TB = 4


def _transpose16(regs, perm, masks):
    for k in range(4):
        d = 1 << k
        out = list(regs)
        for i in range(LANES):
            if i & d:
                continue
            j = i | d
            a, b = regs[i], regs[j]
            bp = b.at[perm[k]].get(mode="promise_in_bounds")
            ap = a.at[perm[k]].get(mode="promise_in_bounds")
            out[i] = jnp.where(masks[k], bp, a)
            out[j] = jnp.where(masks[k], b, ap)
        regs = out
    return regs


def _body(idx_hbm, w_hbm, out_hbm, idx_v, bufs, trans, sem_g, sem_o):
    wid = lax.axis_index("s") * NC + lax.axis_index("c")
    p0 = wid * PAIRS_W
    pltpu.sync_copy(idx_hbm.at[pl.ds(p0, PAIRS_W)], idx_v)

    lane = lax.iota(jnp.int32, LANES)
    perm = [lax.bitwise_xor(lane, 1 << k) for k in range(4)]
    masks = [lax.bitwise_and(lane, 1 << k) != 0 for k in range(4)]

    for i in range(GB):
        pltpu.async_copy(
            w_hbm.at[idx_v.at[i]], bufs.at[pl.ds(i * 128, 128)], sem_g.at[i]
        )

    def pair_step(t, carry):
        b = lax.bitwise_and(t, GB - 1)
        tb = lax.bitwise_and(t, TB - 1)
        p = p0 + t
        c = lax.shift_right_logical(p, 7)
        j = lax.bitwise_and(p, 127)
        brow = b * 128

        pltpu.make_async_copy(
            w_hbm.at[pl.ds(0, 128)], bufs.at[pl.ds(brow, 128)], sem_g.at[b]
        ).wait()

        @pl.when(t >= TB)
        def _():
            for a in range(EMB_DIM // 8):
                pltpu.make_async_copy(
                    out_hbm.at[0, 0, 0],
                    trans.at[tb, a],
                    sem_o.at[tb],
                ).wait()

        for bt in range(128 // LANES):
            for h in range(EMB_DIM // LANES):
                regs = [
                    bufs[brow + bt * LANES + i, pl.ds(h * LANES, LANES)]
                    for i in range(LANES)
                ]
                regs = _transpose16(regs, perm, masks)
                for fl in range(LANES):
                    f = h * LANES + fl
                    trans[tb, f >> 3, f & 7, pl.ds(bt * LANES, LANES)] = regs[fl]

        for a in range(EMB_DIM // 8):
            pltpu.async_copy(
                trans.at[tb, a],
                out_hbm.at[c, a, j],
                sem_o.at[tb],
            )

        @pl.when(t + GB < PAIRS_W)
        def _():
            pltpu.async_copy(
                w_hbm.at[idx_v.at[t + GB]],
                bufs.at[pl.ds(brow, 128)],
                sem_g.at[b],
            )

        return carry

    lax.fori_loop(0, PAIRS_W, pair_step, 0)

    for tslot in range(TB):
        for a in range(EMB_DIM // 8):
            pltpu.make_async_copy(
                out_hbm.at[0, 0, 0],
                trans.at[tslot, a],
                sem_o.at[tslot],
            ).wait()


NUM_EMB = 1000000
TC_BLK = 32768
QUART = TC_BLK // 4
N_BLK = (NUM_EMB + TC_BLK - 1) // TC_BLK
NUM_EMB_PAD = N_BLK * TC_BLK


def _tc_transpose_body(wt_ref, out_ref):
    for q in range(4):
        out_ref[:, q * EMB_DIM:(q + 1) * EMB_DIM] = (
            wt_ref[:, q * QUART:(q + 1) * QUART].T
        )


def _to_row_major(weight):
    wt = weight.T
    return pl.pallas_call(
        _tc_transpose_body,
        grid=(N_BLK,),
        in_specs=[pl.BlockSpec((EMB_DIM, TC_BLK), lambda j: (0, j))],
        out_specs=pl.BlockSpec((QUART, 128), lambda j: (j, 0)),
        out_shape=jax.ShapeDtypeStruct(
            (NUM_EMB_PAD * EMB_DIM // 128, 128), jnp.float32
        ),
    )(wt)


def _swizzle(tid):
    qshift = QUART.bit_length() - 1
    blk = jnp.bitwise_and(tid, jnp.int32(~(TC_BLK - 1)))
    r = jnp.bitwise_and(tid, QUART - 1)
    q = jnp.bitwise_and(jnp.right_shift(tid, qshift), 3)
    return blk + jnp.left_shift(r, 2) + q


@jax.jit
def _gather(idx, weight):
    mesh = plsc.VectorSubcoreMesh(core_axis_name="c", subcore_axis_name="s")
    fn = pl.kernel(
        _body,
        out_type=jax.ShapeDtypeStruct(
            (N_COLS, EMB_DIM // 8, N_ROWS // 128, 8, 128), jnp.float32
        ),
        mesh=mesh,
        scratch_types=[
            pltpu.VMEM((PAIRS_W, 128), jnp.int32),
            pltpu.VMEM((GB * 128, EMB_DIM), jnp.float32),
            pltpu.VMEM((TB, EMB_DIM // 8, 8, 128), jnp.float32),
            pltpu.SemaphoreType.DMA((GB,)),
            pltpu.SemaphoreType.DMA((TB,)),
        ],
        compiler_params=pltpu.CompilerParams(use_tc_tiling_on_sc=False),
    )
    return fn(idx, weight)


def kernel(token_ids, weight):
    tid2 = _swizzle(token_ids.astype(jnp.int32)).T.reshape(N_PAIRS, 128)
    w_rm = _to_row_major(weight).reshape(NUM_EMB_PAD, EMB_DIM)
    out5 = _gather(tid2, w_rm)
    return out5.transpose(2, 4, 0, 1, 3).reshape(N_ROWS, N_COLS, EMB_DIM)
